# baseline (speedup 1.0000x reference)
import functools

import jax
import jax.numpy as jnp
from jax import lax
from jax.experimental import pallas as pl
from jax.experimental.pallas import tpu as pltpu
from jax.experimental.pallas import tpu_sc as plsc

B = 16384
D = 32
V = 1_000_000

_NC, _NS = 2, 16
_NW = _NC * _NS
_RPW = V // _NW
_CW = 384
_NCH = 84
_ND = 3
_RG = 16
_NG = 8
_FE = 999_936
_HCAP = 768
_TRASH = B
_DIAG_NO_EXTRACT = False


def _emit_pass(ids_hbm, t1_hbm, t2_hbm, tl1_hbm, tl2_hbm, out_hbm,
               idsv, cb1, cb2, tl1, tl2, hit_ids, hit_pos, exi, exq,
               outring, idxring, sem1, sem2, s_sc, lo, hi):
    i16 = lax.iota(jnp.int32, 16)

    a0 = (lo // _CW) * _CW

    def chunk_base(t):
        return pl.multiple_of(jnp.minimum(a0 + t * _CW, _FE - _CW), 128)

    def issue_chunk(b, bt):
        for r in range(D // 8):
            rs = pl.ds(r * 8, 8)
            pltpu.async_copy(t1_hbm.at[rs, pl.ds(bt, _CW)],
                             cb1.at[b, rs, :], sem1[b])
            pltpu.async_copy(t2_hbm.at[rs, pl.ds(bt, _CW)],
                             cb2.at[b, rs, :], sem2[b])

    for t0 in range(_ND - 1):
        issue_chunk(t0, chunk_base(t0))

    pltpu.sync_copy(ids_hbm.at[pl.ds(0, B)], idsv)
    pltpu.sync_copy(tl1_hbm, tl1)
    pltpu.sync_copy(tl2_hbm, tl2)

    lov = jnp.full((16,), lo, jnp.int32)
    trv = jnp.full((16,), _TRASH, jnp.int32)
    for q in range(_HCAP // 16):
        hit_ids[pl.ds(q * 16, 16)] = lov
        hit_pos[pl.ds(q * 16, 16)] = trv

    def scan_body(v, cnt):
        for qq in range(4):
            base = (v * 4 + qq) * 16
            idv = idsv[pl.ds(base, 16)]
            m = (idv >= lo) & (idv < hi)
            plsc.store_compressed(hit_ids.at[pl.ds(cnt, 16)], idv, mask=m)
            plsc.store_compressed(hit_pos.at[pl.ds(cnt, 16)], i16 + base, mask=m)
            cnt = jnp.minimum(cnt + jnp.sum(m.astype(jnp.int32)), _HCAP - 16)
        return cnt

    cnt = lax.fori_loop(0, B // 64, scan_body, 0)
    nv = (cnt + 15) // 16

    def extract(r1, r2, pref, base_c, wlo, whi, carry):
        def sub(k, c2):
            hid = hit_ids[pl.ds(k * 16, 16)]
            hpv = hit_pos[pl.ds(k * 16, 16)]
            m = (hid >= wlo) & (hid < whi)
            nx = jnp.sum(m.astype(jnp.int32))
            plsc.store_compressed(exi.at[pl.ds(0, 16)], hid, mask=m)
            plsc.store_compressed(exq.at[pl.ds(0, 16)], hpv, mask=m)
            exiv = exi[pl.ds(0, 16)]
            exqv = exq[pl.ds(0, 16)]

            def hitloop(j, c3):
                cnt_e, g = c3
                sel = i16 == j
                rel = jnp.sum(jnp.where(sel, exiv, 0)) - base_c
                bpos = jnp.sum(jnp.where(sel, exqv, 0))
                colv = jnp.full((16,), rel, jnp.int32)
                slot = cnt_e % (_NG * _RG)
                half = slot // _RG
                slotin = slot % _RG
                g1 = plsc.load_gather(r1, pref + [i16, colv])
                g2 = plsc.load_gather(r1, pref + [i16 + 16, colv])
                g3 = plsc.load_gather(r2, pref + [i16, colv])
                g4 = plsc.load_gather(r2, pref + [i16 + 16, colv])
                outring[slot, pl.ds(0, 16)] = g1
                outring[slot, pl.ds(16, 16)] = g2
                outring[slot, pl.ds(32, 16)] = g3
                outring[slot, pl.ds(48, 16)] = g4
                plsc.store_scatter(idxring.at[half],
                                   [jnp.full((16,), slotin, jnp.int32)],
                                   jnp.full((16,), bpos, jnp.int32),
                                   mask=i16 == 0)
                ncnt = cnt_e + 1

                @pl.when(ncnt % _RG == 0)
                def _flush():
                    pltpu.async_copy(outring.at[pl.ds(half * _RG, _RG)],
                                     out_hbm.at[idxring.at[half]], s_sc)

                    @pl.when(g >= _NG - 1)
                    def _drain():
                        pltpu.make_async_copy(
                            out_hbm.at[pl.ds(0, _RG)],
                            outring.at[pl.ds(0, _RG)], s_sc).wait()

                return ncnt, g + (ncnt % _RG == 0).astype(jnp.int32)

            return lax.fori_loop(0, nx, hitloop, c2)

        return lax.fori_loop(0, nv, sub, carry)

    def chunk_body(t3, carry):
        for b in range(_ND):
            t = t3 * _ND + b
            base_c = chunk_base(t)
            pltpu.make_async_copy(t1_hbm.at[:, pl.ds(0, _CW)],
                                  cb1.at[b], sem1[b]).wait()
            pltpu.make_async_copy(t2_hbm.at[:, pl.ds(0, _CW)],
                                  cb2.at[b], sem2[b]).wait()

            @pl.when(t + _ND - 1 < _NCH)
            def _issue_next(t=t, b=b):
                issue_chunk((b + _ND - 1) % _ND, chunk_base(t + _ND - 1))

            wlo = jnp.maximum(base_c, lo)
            whi = jnp.minimum(base_c + _CW, hi)
            pv = jnp.full((16,), b, jnp.int32)
            if not _DIAG_NO_EXTRACT:
                carry = extract(cb1, cb2, [pv], base_c, wlo, whi, carry)
        return carry

    cnt_e, g = lax.fori_loop(0, _NCH // _ND, chunk_body, (0, 0))

    cnt_e, g = extract(tl1, tl2, [], _FE, jnp.maximum(_FE, lo),
                       jnp.minimum(V, hi), (cnt_e, g))

    slotin_f = cnt_e % _RG
    half_f = (cnt_e // _RG) % _NG
    for q in range(_RG // 16):
        lanes = i16 + q * 16
        cur = idxring[half_f, pl.ds(q * 16, 16)]
        idxring[half_f, pl.ds(q * 16, 16)] = jnp.where(
            lanes >= slotin_f, jnp.full((16,), _TRASH, jnp.int32), cur)
    pltpu.async_copy(outring.at[pl.ds(half_f * _RG, _RG)],
                     out_hbm.at[idxring.at[half_f]], s_sc)
    pltpu.make_async_copy(out_hbm.at[pl.ds(0, _RG)],
                          outring.at[pl.ds(0, _RG)], s_sc).wait()

    for k in range(1, _NG):
        @pl.when(g >= k)
        def _final_drain():
            pltpu.make_async_copy(out_hbm.at[pl.ds(0, _RG)],
                                  outring.at[pl.ds(0, _RG)], s_sc).wait()


def _sc_body(uid, iid, guT, giT, muT, miT, tgu, tgi, tmu, tmi,
             out_u, out_i,
             idsv, cb1, cb2, tl1, tl2, hit_ids, hit_pos, exi, exq,
             outring, idxring, s10, s11, s12, s20, s21, s22, s_sc):
    wid = lax.axis_index("s") * _NC + lax.axis_index("c")
    lo = wid * _RPW
    hi = lo + _RPW
    args = (idsv, cb1, cb2, tl1, tl2, hit_ids, hit_pos, exi, exq,
            outring, idxring, [s10, s11, s12], [s20, s21, s22], s_sc, lo, hi)
    _emit_pass(uid, guT, muT, tgu, tmu, out_u, *args)
    _emit_pass(iid, giT, miT, tgi, tmi, out_i, *args)


@functools.lru_cache(maxsize=1)
def _build_sc_sweep():
    return functools.partial(
        pl.kernel,
        mesh=plsc.VectorSubcoreMesh(core_axis_name="c", subcore_axis_name="s",
                                    num_cores=_NC),
        out_type=[jax.ShapeDtypeStruct((B + 64, 128), jnp.float32)] * 2,
        scratch_types=[
            pltpu.VMEM((B,), jnp.int32),
            pltpu.VMEM((_ND, D, _CW), jnp.float32),
            pltpu.VMEM((_ND, D, _CW), jnp.float32),
            pltpu.VMEM((D, 64), jnp.float32),
            pltpu.VMEM((D, 64), jnp.float32),
            pltpu.VMEM((_HCAP,), jnp.int32),
            pltpu.VMEM((_HCAP,), jnp.int32),
            pltpu.VMEM((16,), jnp.int32),
            pltpu.VMEM((16,), jnp.int32),
            pltpu.VMEM((_NG * _RG, 128), jnp.float32),
            pltpu.VMEM((_NG, _RG), jnp.int32),
            pltpu.SemaphoreType.DMA,
            pltpu.SemaphoreType.DMA,
            pltpu.SemaphoreType.DMA,
            pltpu.SemaphoreType.DMA,
            pltpu.SemaphoreType.DMA,
            pltpu.SemaphoreType.DMA,
            pltpu.SemaphoreType.DMA,
        ],
        compiler_params=pltpu.CompilerParams(needs_layout_passes=False),
    )(_sc_body)


_BB = 2048


def _mlp_body(u, it, w1u, w1i, b1, w2, b2, w3, b3, wog, woh, bo, out):
    mu = u[:, 32:64]
    mi = it[:, 32:64]
    gmf = u[:, 0:32] * it[:, 0:32]
    h = jnp.dot(mu, w1u[...], preferred_element_type=jnp.float32)
    h = h + jnp.dot(mi, w1i[...], preferred_element_type=jnp.float32)
    h = jnp.maximum(h + b1[...], 0.0)
    h = jnp.maximum(jnp.dot(h, w2[...], preferred_element_type=jnp.float32) + b2[...], 0.0)
    h = jnp.maximum(jnp.dot(h, w3[...], preferred_element_type=jnp.float32) + b3[...], 0.0)
    o = jnp.sum(gmf * wog[...], axis=1) + jnp.sum(h * woh[...], axis=1)
    out[...] = o + bo[0, 0]


def _mlp_call(pu, pi, w1u, w1i, b1, w2, b2, w3, b3, wog, woh, bo):
    nb = B // _BB
    row = lambda i: (i, 0)
    rep = lambda i: (0, 0)
    return pl.pallas_call(
        _mlp_body,
        grid=(nb,),
        in_specs=[
            pl.BlockSpec((_BB, 128), row),
            pl.BlockSpec((_BB, 128), row),
            pl.BlockSpec((D, 64), rep),
            pl.BlockSpec((D, 64), rep),
            pl.BlockSpec((1, 64), rep),
            pl.BlockSpec((64, 32), rep),
            pl.BlockSpec((1, 32), rep),
            pl.BlockSpec((32, 16), rep),
            pl.BlockSpec((1, 16), rep),
            pl.BlockSpec((1, D), rep),
            pl.BlockSpec((1, 16), rep),
            pl.BlockSpec((1, 1), rep),
        ],
        out_specs=pl.BlockSpec((_BB,), lambda i: (i,)),
        out_shape=jax.ShapeDtypeStruct((B,), jnp.float32),
        compiler_params=pltpu.CompilerParams(
            dimension_semantics=("arbitrary",),
        ),
    )(pu, pi, w1u, w1i, b1, w2, b2, w3, b3, wog, woh, bo)


def kernel(user_ids, item_ids, gmf_user, gmf_item, mlp_user, mlp_item,
           W1, b1, W2, b2, W3, b3, Wo, bo):
    uid = user_ids.astype(jnp.int32)
    iid = item_ids.astype(jnp.int32)
    pu, pi = _build_sc_sweep()(
        uid, iid,
        gmf_user.T, gmf_item.T, mlp_user.T, mlp_item.T,
        gmf_user[_FE:].T, gmf_item[_FE:].T,
        mlp_user[_FE:].T, mlp_item[_FE:].T,
    )
    return _mlp_call(
        pu, pi,
        W1[:D], W1[D:], b1.reshape(1, 64),
        W2, b2.reshape(1, 32),
        W3, b3.reshape(1, 16),
        Wo[:D].reshape(1, D), Wo[D:].reshape(1, 16),
        bo.reshape(1, 1),
    )


# speedup vs baseline: 1.0212x; 1.0212x over previous
import functools

import jax
import jax.numpy as jnp
from jax import lax
from jax.experimental import pallas as pl
from jax.experimental.pallas import tpu as pltpu
from jax.experimental.pallas import tpu_sc as plsc

B = 16384
D = 32
V = 1_000_000

_NC, _NS = 2, 16
_NW = _NC * _NS
_RPW = V // _NW
_CW = 384
_NCH = 83
_ND = 3
_RG = 16
_NG = 8
_FE = 999_936
_HCAP = 768
_TRASH = B


def _emit_pass(ids_hbm, t1_hbm, t2_hbm, tl1_hbm, tl2_hbm, out_hbm,
               idsv, cb1, cb2, tl1, tl2, hit_ids, hit_pos, exi, exq,
               outring, idxring, sem1, sem2, s_sc, lo, hi):
    i16 = lax.iota(jnp.int32, 16)

    a0 = (lo // _CW) * _CW

    def chunk_base(t):
        return pl.multiple_of(jnp.minimum(a0 + t * _CW, _FE - _CW), 128)

    def issue_chunk(b, bt):
        for r in range(D // 8):
            rs = pl.ds(r * 8, 8)
            pltpu.async_copy(t1_hbm.at[rs, pl.ds(bt, _CW)],
                             cb1.at[b, rs, :], sem1[b])
            pltpu.async_copy(t2_hbm.at[rs, pl.ds(bt, _CW)],
                             cb2.at[b, rs, :], sem2[b])

    for t0 in range(_ND - 1):
        issue_chunk(t0, chunk_base(t0))

    pltpu.sync_copy(ids_hbm.at[pl.ds(0, B)], idsv)
    pltpu.sync_copy(tl1_hbm, tl1)
    pltpu.sync_copy(tl2_hbm, tl2)

    lov = jnp.full((16,), lo, jnp.int32)
    trv = jnp.full((16,), _TRASH, jnp.int32)
    for q in range(_HCAP // 16):
        hit_ids[pl.ds(q * 16, 16)] = lov
        hit_pos[pl.ds(q * 16, 16)] = trv

    def scan_body(v, cnt):
        for qq in range(4):
            base = (v * 4 + qq) * 16
            idv = idsv[pl.ds(base, 16)]
            m = (idv >= lo) & (idv < hi)
            plsc.store_compressed(hit_ids.at[pl.ds(cnt, 16)], idv, mask=m)
            plsc.store_compressed(hit_pos.at[pl.ds(cnt, 16)], i16 + base, mask=m)
            cnt = jnp.minimum(cnt + jnp.sum(m.astype(jnp.int32)), _HCAP - 16)
        return cnt

    cnt = lax.fori_loop(0, B // 64, scan_body, 0)
    nv = (cnt + 15) // 16

    def extract(r1, r2, pref, base_c, wlo, whi, carry):
        def sub(k, c2):
            hid = hit_ids[pl.ds(k * 16, 16)]
            hpv = hit_pos[pl.ds(k * 16, 16)]
            m = (hid >= wlo) & (hid < whi)
            nx = jnp.sum(m.astype(jnp.int32))
            plsc.store_compressed(exi.at[pl.ds(0, 16)], hid, mask=m)
            plsc.store_compressed(exq.at[pl.ds(0, 16)], hpv, mask=m)
            exiv = exi[pl.ds(0, 16)]
            exqv = exq[pl.ds(0, 16)]

            def hitloop(j, c3):
                cnt_e, g = c3
                sel = i16 == j
                rel = jnp.sum(jnp.where(sel, exiv, 0)) - base_c
                bpos = jnp.sum(jnp.where(sel, exqv, 0))
                colv = jnp.full((16,), rel, jnp.int32)
                slot = cnt_e % (_NG * _RG)
                half = slot // _RG
                slotin = slot % _RG
                g1 = plsc.load_gather(r1, pref + [i16, colv])
                g2 = plsc.load_gather(r1, pref + [i16 + 16, colv])
                g3 = plsc.load_gather(r2, pref + [i16, colv])
                g4 = plsc.load_gather(r2, pref + [i16 + 16, colv])
                outring[slot, pl.ds(0, 16)] = g1
                outring[slot, pl.ds(16, 16)] = g2
                outring[slot, pl.ds(32, 16)] = g3
                outring[slot, pl.ds(48, 16)] = g4
                plsc.store_scatter(idxring.at[half],
                                   [jnp.full((16,), slotin, jnp.int32)],
                                   jnp.full((16,), bpos, jnp.int32),
                                   mask=i16 == 0)
                ncnt = cnt_e + 1

                @pl.when(ncnt % _RG == 0)
                def _flush():
                    pltpu.async_copy(outring.at[pl.ds(half * _RG, _RG)],
                                     out_hbm.at[idxring.at[half]], s_sc)

                    @pl.when(g >= _NG - 1)
                    def _drain():
                        pltpu.make_async_copy(
                            out_hbm.at[pl.ds(0, _RG)],
                            outring.at[pl.ds(0, _RG)], s_sc).wait()

                return ncnt, g + (ncnt % _RG == 0).astype(jnp.int32)

            return lax.fori_loop(0, nx, hitloop, c2)

        return lax.fori_loop(0, nv, sub, carry)

    def chunk_body(t, carry):
        p = t % _ND
        base_c = chunk_base(t)
        for b in range(_ND):
            @pl.when(p == b)
            def _wait(b=b):
                pltpu.make_async_copy(t1_hbm.at[:, pl.ds(0, _CW)],
                                      cb1.at[b], sem1[b]).wait()
                pltpu.make_async_copy(t2_hbm.at[:, pl.ds(0, _CW)],
                                      cb2.at[b], sem2[b]).wait()

        @pl.when(t + _ND - 1 < _NCH)
        def _issue_next():
            bn = chunk_base(t + _ND - 1)
            for b in range(_ND):
                @pl.when((t + _ND - 1) % _ND == b)
                def _iss(b=b):
                    issue_chunk(b, bn)

        wlo = jnp.maximum(base_c, lo)
        whi = jnp.minimum(base_c + _CW, hi)
        pv = jnp.full((16,), p, jnp.int32)
        return extract(cb1, cb2, [pv], base_c, wlo, whi, carry)

    cnt_e, g = lax.fori_loop(0, _NCH, chunk_body, (0, 0))

    cnt_e, g = extract(tl1, tl2, [], _FE, jnp.maximum(_FE, lo),
                       jnp.minimum(V, hi), (cnt_e, g))

    slotin_f = cnt_e % _RG
    half_f = (cnt_e // _RG) % _NG
    for q in range(_RG // 16):
        lanes = i16 + q * 16
        cur = idxring[half_f, pl.ds(q * 16, 16)]
        idxring[half_f, pl.ds(q * 16, 16)] = jnp.where(
            lanes >= slotin_f, jnp.full((16,), _TRASH, jnp.int32), cur)
    pltpu.async_copy(outring.at[pl.ds(half_f * _RG, _RG)],
                     out_hbm.at[idxring.at[half_f]], s_sc)
    pltpu.make_async_copy(out_hbm.at[pl.ds(0, _RG)],
                          outring.at[pl.ds(0, _RG)], s_sc).wait()

    for k in range(1, _NG):
        @pl.when(g >= k)
        def _final_drain():
            pltpu.make_async_copy(out_hbm.at[pl.ds(0, _RG)],
                                  outring.at[pl.ds(0, _RG)], s_sc).wait()


def _sc_body(uid, iid, guT, giT, muT, miT, tgu, tgi, tmu, tmi,
             out_u, out_i,
             idsv, cb1, cb2, tl1, tl2, hit_ids, hit_pos, exi, exq,
             outring, idxring, s10, s11, s12, s20, s21, s22, s_sc):
    wid = lax.axis_index("s") * _NC + lax.axis_index("c")
    lo = wid * _RPW
    hi = lo + _RPW
    args = (idsv, cb1, cb2, tl1, tl2, hit_ids, hit_pos, exi, exq,
            outring, idxring, [s10, s11, s12], [s20, s21, s22], s_sc, lo, hi)
    _emit_pass(uid, guT, muT, tgu, tmu, out_u, *args)
    _emit_pass(iid, giT, miT, tgi, tmi, out_i, *args)


@functools.lru_cache(maxsize=1)
def _build_sc_sweep():
    return functools.partial(
        pl.kernel,
        mesh=plsc.VectorSubcoreMesh(core_axis_name="c", subcore_axis_name="s",
                                    num_cores=_NC),
        out_type=[jax.ShapeDtypeStruct((B + 64, 128), jnp.float32)] * 2,
        scratch_types=[
            pltpu.VMEM((B,), jnp.int32),
            pltpu.VMEM((_ND, D, _CW), jnp.float32),
            pltpu.VMEM((_ND, D, _CW), jnp.float32),
            pltpu.VMEM((D, 64), jnp.float32),
            pltpu.VMEM((D, 64), jnp.float32),
            pltpu.VMEM((_HCAP,), jnp.int32),
            pltpu.VMEM((_HCAP,), jnp.int32),
            pltpu.VMEM((16,), jnp.int32),
            pltpu.VMEM((16,), jnp.int32),
            pltpu.VMEM((_NG * _RG, 128), jnp.float32),
            pltpu.VMEM((_NG, _RG), jnp.int32),
            pltpu.SemaphoreType.DMA,
            pltpu.SemaphoreType.DMA,
            pltpu.SemaphoreType.DMA,
            pltpu.SemaphoreType.DMA,
            pltpu.SemaphoreType.DMA,
            pltpu.SemaphoreType.DMA,
            pltpu.SemaphoreType.DMA,
        ],
        compiler_params=pltpu.CompilerParams(needs_layout_passes=False),
    )(_sc_body)


_BB = 2048


def _mlp_body(u, it, w1u, w1i, b1, w2, b2, w3, b3, wog, woh, bo, out):
    mu = u[:, 32:64]
    mi = it[:, 32:64]
    gmf = u[:, 0:32] * it[:, 0:32]
    h = jnp.dot(mu, w1u[...], preferred_element_type=jnp.float32)
    h = h + jnp.dot(mi, w1i[...], preferred_element_type=jnp.float32)
    h = jnp.maximum(h + b1[...], 0.0)
    h = jnp.maximum(jnp.dot(h, w2[...], preferred_element_type=jnp.float32) + b2[...], 0.0)
    h = jnp.maximum(jnp.dot(h, w3[...], preferred_element_type=jnp.float32) + b3[...], 0.0)
    o = jnp.sum(gmf * wog[...], axis=1) + jnp.sum(h * woh[...], axis=1)
    out[...] = o + bo[0, 0]


def _mlp_call(pu, pi, w1u, w1i, b1, w2, b2, w3, b3, wog, woh, bo):
    nb = B // _BB
    row = lambda i: (i, 0)
    rep = lambda i: (0, 0)
    return pl.pallas_call(
        _mlp_body,
        grid=(nb,),
        in_specs=[
            pl.BlockSpec((_BB, 128), row),
            pl.BlockSpec((_BB, 128), row),
            pl.BlockSpec((D, 64), rep),
            pl.BlockSpec((D, 64), rep),
            pl.BlockSpec((1, 64), rep),
            pl.BlockSpec((64, 32), rep),
            pl.BlockSpec((1, 32), rep),
            pl.BlockSpec((32, 16), rep),
            pl.BlockSpec((1, 16), rep),
            pl.BlockSpec((1, D), rep),
            pl.BlockSpec((1, 16), rep),
            pl.BlockSpec((1, 1), rep),
        ],
        out_specs=pl.BlockSpec((_BB,), lambda i: (i,)),
        out_shape=jax.ShapeDtypeStruct((B,), jnp.float32),
        compiler_params=pltpu.CompilerParams(
            dimension_semantics=("arbitrary",),
        ),
    )(pu, pi, w1u, w1i, b1, w2, b2, w3, b3, wog, woh, bo)


def kernel(user_ids, item_ids, gmf_user, gmf_item, mlp_user, mlp_item,
           W1, b1, W2, b2, W3, b3, Wo, bo):
    uid = user_ids.astype(jnp.int32)
    iid = item_ids.astype(jnp.int32)
    pu, pi = _build_sc_sweep()(
        uid, iid,
        gmf_user.T, gmf_item.T, mlp_user.T, mlp_item.T,
        gmf_user[_FE:].T, gmf_item[_FE:].T,
        mlp_user[_FE:].T, mlp_item[_FE:].T,
    )
    return _mlp_call(
        pu, pi,
        W1[:D], W1[D:], b1.reshape(1, 64),
        W2, b2.reshape(1, 32),
        W3, b3.reshape(1, 16),
        Wo[:D].reshape(1, D), Wo[D:].reshape(1, 16),
        bo.reshape(1, 1),
    )
